# baseline (speedup 1.0000x reference)
import jax
import jax.numpy as jnp
from jax import lax
from jax.experimental import pallas as pl
from jax.experimental.pallas import tpu as pltpu
from jax.experimental.pallas import tpu_sc as plsc

NUM_NODES = 1000000
MEMORY_DIM = 64
BATCH = 16384

_NC = 2
_NS = 16
_NW = _NC * _NS
_BPT = BATCH // _NW
_JCHUNK = 16


_mesh = plsc.VectorSubcoreMesh(core_axis_name="c", subcore_axis_name="s")


@pl.kernel(
    out_type=(
        jax.ShapeDtypeStruct((MEMORY_DIM, BATCH), jnp.float32),
        jax.ShapeDtypeStruct((BATCH,), jnp.int32),
    ),
    mesh=_mesh,
    compiler_params=pltpu.CompilerParams(use_tc_tiling_on_sc=False),
    scratch_types=[
        pltpu.VMEM((MEMORY_DIM, _BPT), jnp.int32),
        pltpu.VMEM((MEMORY_DIM, _BPT), jnp.float32),
        pltpu.VMEM((_BPT,), jnp.int32),
        pltpu.VMEM((_BPT,), jnp.int32),
        pltpu.SemaphoreType.DMA,
        pltpu.SemaphoreType.DMA,
    ],
)
def _gather_kernel(idx_all_hbm, mem1d_hbm, n_id_hbm, last_hbm,
                   outt_hbm, last_out_hbm,
                   idxs_v, cols_v, idx_v, last_v, sem_cols, sem_last):
    wid = lax.axis_index("s") * _NC + lax.axis_index("c")
    base = wid * _BPT
    pltpu.sync_copy(idx_all_hbm.at[:, pl.ds(base, _BPT)], idxs_v)
    pltpu.sync_copy(n_id_hbm.at[pl.ds(base, _BPT)], idx_v)
    cp_last = pltpu.async_copy(last_hbm.at[idx_v], last_v, sem_last)

    def chunk(g):
        copies = []
        for jc in range(_JCHUNK):
            j = g * _JCHUNK + jc
            copies.append(pltpu.async_copy(
                mem1d_hbm.at[idxs_v.at[j]], cols_v.at[j], sem_cols))
        for cp in copies:
            cp.wait()

    pl.loop(0, MEMORY_DIM // _JCHUNK)(chunk)

    cp_last.wait()
    pltpu.sync_copy(last_v, last_out_hbm.at[pl.ds(base, _BPT)])
    pltpu.sync_copy(cols_v, outt_hbm.at[:, pl.ds(base, _BPT)])


def kernel(n_id, memory, last_update):
    mem1d = memory.T.reshape(-1)
    idx_all = n_id[None, :] + (
        jnp.arange(MEMORY_DIM, dtype=jnp.int32) * NUM_NODES)[:, None]
    outt, last_out = _gather_kernel(idx_all, mem1d, n_id, last_update)
    return (outt.T, last_out, jnp.array(0, dtype=jnp.int32))


# speedup vs baseline: 14.8483x; 14.8483x over previous
import jax
import jax.numpy as jnp
from jax import lax
from jax.experimental import pallas as pl
from jax.experimental.pallas import tpu as pltpu
from jax.experimental.pallas import tpu_sc as plsc

NUM_NODES = 1000000
MEMORY_DIM = 64
BATCH = 16384

_NC = 2
_NS = 16
_NW = _NC * _NS
_BPT = BATCH // _NW
_G = 16


_mesh = plsc.VectorSubcoreMesh(core_axis_name="c", subcore_axis_name="s")


@pl.kernel(
    out_type=(
        jax.ShapeDtypeStruct((MEMORY_DIM, BATCH), jnp.float32),
        jax.ShapeDtypeStruct((BATCH,), jnp.int32),
    ),
    mesh=_mesh,
    compiler_params=pltpu.CompilerParams(needs_layout_passes=False),
    scratch_types=[
        pltpu.VMEM((_BPT,), jnp.int32),
        pltpu.VMEM((MEMORY_DIM, _BPT), jnp.float32),
        pltpu.VMEM((MEMORY_DIM, 128), jnp.float32),
        pltpu.VMEM((MEMORY_DIM, 128), jnp.float32),
        pltpu.VMEM((_BPT,), jnp.int32),
        pltpu.SemaphoreType.DMA,
        pltpu.SemaphoreType.DMA,
        pltpu.SemaphoreType.DMA,
    ],
)
def _gather_kernel(n_id_hbm, memt_hbm, last_hbm, outt_hbm, last_out_hbm,
                   idx_v, stag_v, buf0_v, buf1_v, last_v,
                   sem0, sem1, sem_last):
    wid = lax.axis_index("s") * _NC + lax.axis_index("c")
    base = wid * _BPT
    pltpu.sync_copy(n_id_hbm.at[pl.ds(base, _BPT)], idx_v)
    cp_last = pltpu.async_copy(last_hbm.at[idx_v], last_v, sem_last)

    bufs = (buf0_v, buf1_v)
    sems = (sem0, sem1)
    lanes = lax.iota(jnp.int32, 16)
    jvecs = [lanes + 16 * k for k in range(MEMORY_DIM // 16)]

    def fetch(i, slot):
        blk = pl.multiple_of((i >> 7) << 7, 128)
        return pltpu.async_copy(
            memt_hbm.at[:, pl.ds(blk, 128)], bufs[slot], sems[slot])

    def serve(off, r, slot):
        offv = jnp.full((16,), off, dtype=jnp.int32)
        rv = jnp.full((16,), r, dtype=jnp.int32)
        for jv in jvecs:
            vals = plsc.load_gather(bufs[slot], [jv, offv])
            plsc.store_scatter(stag_v, [jv, rv], vals)

    def group(g):
        iv = plsc.load_gather(idx_v, [lanes + g * _G])
        copies = [None, None]
        for k in range(_G):
            slot = k % 2
            if copies[slot] is not None:
                copies[slot].wait()
                serve(iv[k - 2] & 127, g * _G + k - 2, slot)
            copies[slot] = fetch(iv[k], slot)
        for k in (_G - 2, _G - 1):
            slot = k % 2
            copies[slot].wait()
            serve(iv[k] & 127, g * _G + k, slot)

    pl.loop(0, _BPT // _G)(group)

    cp_last.wait()
    pltpu.sync_copy(last_v, last_out_hbm.at[pl.ds(base, _BPT)])
    pltpu.sync_copy(stag_v, outt_hbm.at[:, pl.ds(base, _BPT)])


def kernel(n_id, memory, last_update):
    outt, last_out = _gather_kernel(n_id, memory.T, last_update)
    return (outt.T, last_out, jnp.array(0, dtype=jnp.int32))


# speedup vs baseline: 19.0363x; 1.2821x over previous
import jax
import jax.numpy as jnp
from jax import lax
from jax.experimental import pallas as pl
from jax.experimental.pallas import tpu as pltpu
from jax.experimental.pallas import tpu_sc as plsc

NUM_NODES = 1000000
MEMORY_DIM = 64
BATCH = 16384

_NC = 2
_NS = 16
_NW = _NC * _NS
_BPT = BATCH // _NW
_G = 16


_mesh = plsc.VectorSubcoreMesh(core_axis_name="c", subcore_axis_name="s")


@pl.kernel(
    out_type=(
        jax.ShapeDtypeStruct((MEMORY_DIM, BATCH), jnp.float32),
        jax.ShapeDtypeStruct((BATCH,), jnp.int32),
    ),
    mesh=_mesh,
    compiler_params=pltpu.CompilerParams(needs_layout_passes=False),
    scratch_types=[
        pltpu.VMEM((_BPT,), jnp.int32),
        pltpu.VMEM((MEMORY_DIM, _BPT), jnp.float32),
        pltpu.VMEM((MEMORY_DIM, 128), jnp.float32),
        pltpu.VMEM((MEMORY_DIM, 128), jnp.float32),
        pltpu.VMEM((MEMORY_DIM, 128), jnp.float32),
        pltpu.VMEM((MEMORY_DIM, 128), jnp.float32),
        pltpu.VMEM((_BPT,), jnp.int32),
        pltpu.SemaphoreType.DMA,
        pltpu.SemaphoreType.DMA,
        pltpu.SemaphoreType.DMA,
        pltpu.SemaphoreType.DMA,
        pltpu.SemaphoreType.DMA,
    ],
)
def _gather_kernel(n_id_hbm, memt_hbm, last_hbm, outt_hbm, last_out_hbm,
                   idx_v, stag_v, buf0_v, buf1_v, buf2_v, buf3_v, last_v,
                   sem0, sem1, sem2, sem3, sem_last):
    wid = lax.axis_index("s") * _NC + lax.axis_index("c")
    base = wid * _BPT
    pltpu.sync_copy(n_id_hbm.at[pl.ds(base, _BPT)], idx_v)
    cp_last = pltpu.async_copy(last_hbm.at[idx_v], last_v, sem_last)

    bufs = (buf0_v, buf1_v, buf2_v, buf3_v)
    sems = (sem0, sem1, sem2, sem3)
    _D = len(bufs)
    lanes = lax.iota(jnp.int32, 16)
    jvecs = [lanes + 16 * k for k in range(MEMORY_DIM // 16)]

    def fetch(i, slot):
        blk = pl.multiple_of((i >> 7) << 7, 128)
        return pltpu.async_copy(
            memt_hbm.at[:, pl.ds(blk, 128)], bufs[slot], sems[slot])

    def serve(off, r, slot):
        offv = jnp.full((16,), off, dtype=jnp.int32)
        rv = jnp.full((16,), r, dtype=jnp.int32)
        for jv in jvecs:
            vals = plsc.load_gather(bufs[slot], [jv, offv])
            plsc.store_scatter(stag_v, [jv, rv], vals)

    def group(g):
        iv = plsc.load_gather(idx_v, [lanes + g * _G])
        offs = iv & 127
        copies = [None] * _D
        for k in range(_G):
            slot = k % _D
            if copies[slot] is not None:
                copies[slot].wait()
                serve(offs[k - _D], g * _G + k - _D, slot)
            copies[slot] = fetch(iv[k], slot)
        for k in range(_G - _D, _G):
            slot = k % _D
            copies[slot].wait()
            serve(offs[k], g * _G + k, slot)

    pl.loop(0, _BPT // _G)(group)

    cp_last.wait()
    pltpu.sync_copy(last_v, last_out_hbm.at[pl.ds(base, _BPT)])
    pltpu.sync_copy(stag_v, outt_hbm.at[:, pl.ds(base, _BPT)])


def kernel(n_id, memory, last_update):
    outt, last_out = _gather_kernel(n_id, memory.T, last_update)
    return (outt.T, last_out, jnp.array(0, dtype=jnp.int32))


# speedup vs baseline: 21.1836x; 1.1128x over previous
import jax
import jax.numpy as jnp
from jax import lax
from jax.experimental import pallas as pl
from jax.experimental.pallas import tpu as pltpu
from jax.experimental.pallas import tpu_sc as plsc

NUM_NODES = 1000000
MEMORY_DIM = 64
BATCH = 16384

_NC = 2
_NS = 16
_NW = _NC * _NS
_BPT = BATCH // _NW
_G = 16


_mesh = plsc.VectorSubcoreMesh(core_axis_name="c", subcore_axis_name="s")


@pl.kernel(
    out_type=(
        jax.ShapeDtypeStruct((MEMORY_DIM, BATCH), jnp.float32),
        jax.ShapeDtypeStruct((BATCH,), jnp.int32),
    ),
    mesh=_mesh,
    compiler_params=pltpu.CompilerParams(needs_layout_passes=False),
    scratch_types=[
        pltpu.VMEM((_BPT,), jnp.int32),
        pltpu.VMEM((MEMORY_DIM, _BPT), jnp.float32),
        pltpu.VMEM((MEMORY_DIM, 128), jnp.float32),
        pltpu.VMEM((MEMORY_DIM, 128), jnp.float32),
        pltpu.VMEM((MEMORY_DIM, 128), jnp.float32),
        pltpu.VMEM((MEMORY_DIM, 128), jnp.float32),
        pltpu.VMEM((MEMORY_DIM, 128), jnp.float32),
        pltpu.VMEM((MEMORY_DIM, 128), jnp.float32),
        pltpu.VMEM((MEMORY_DIM, 128), jnp.float32),
        pltpu.VMEM((MEMORY_DIM, 128), jnp.float32),
        pltpu.VMEM((_BPT,), jnp.int32),
        pltpu.SemaphoreType.DMA,
        pltpu.SemaphoreType.DMA,
        pltpu.SemaphoreType.DMA,
        pltpu.SemaphoreType.DMA,
        pltpu.SemaphoreType.DMA,
        pltpu.SemaphoreType.DMA,
        pltpu.SemaphoreType.DMA,
        pltpu.SemaphoreType.DMA,
        pltpu.SemaphoreType.DMA,
    ],
)
def _gather_kernel(n_id_hbm, memt_hbm, last_hbm, outt_hbm, last_out_hbm,
                   idx_v, stag_v, buf0_v, buf1_v, buf2_v, buf3_v,
                   buf4_v, buf5_v, buf6_v, buf7_v, last_v,
                   sem0, sem1, sem2, sem3, sem4, sem5, sem6, sem7, sem_last):
    wid = lax.axis_index("s") * _NC + lax.axis_index("c")
    base = wid * _BPT
    pltpu.sync_copy(n_id_hbm.at[pl.ds(base, _BPT)], idx_v)
    cp_last = pltpu.async_copy(last_hbm.at[idx_v], last_v, sem_last)

    bufs = (buf0_v, buf1_v, buf2_v, buf3_v, buf4_v, buf5_v, buf6_v, buf7_v)
    sems = (sem0, sem1, sem2, sem3, sem4, sem5, sem6, sem7)
    _D = len(bufs)
    lanes = lax.iota(jnp.int32, 16)
    jvecs = [lanes + 16 * k for k in range(MEMORY_DIM // 16)]

    def fetch(i, slot):
        blk = pl.multiple_of((i >> 7) << 7, 128)
        return pltpu.async_copy(
            memt_hbm.at[:, pl.ds(blk, 128)], bufs[slot], sems[slot])

    def serve(off, r, slot):
        offv = jnp.full((16,), off, dtype=jnp.int32)
        rv = jnp.full((16,), r, dtype=jnp.int32)
        for jv in jvecs:
            vals = plsc.load_gather(bufs[slot], [jv, offv])
            plsc.store_scatter(stag_v, [jv, rv], vals)

    def group(g):
        iv = plsc.load_gather(idx_v, [lanes + g * _G])
        offs = iv & 127
        copies = [None] * _D
        for k in range(_G):
            slot = k % _D
            if copies[slot] is not None:
                copies[slot].wait()
                serve(offs[k - _D], g * _G + k - _D, slot)
            copies[slot] = fetch(iv[k], slot)
        for k in range(_G - _D, _G):
            slot = k % _D
            copies[slot].wait()
            serve(offs[k], g * _G + k, slot)

    pl.loop(0, _BPT // _G)(group)

    cp_last.wait()
    pltpu.sync_copy(last_v, last_out_hbm.at[pl.ds(base, _BPT)])
    pltpu.sync_copy(stag_v, outt_hbm.at[:, pl.ds(base, _BPT)])


def kernel(n_id, memory, last_update):
    outt, last_out = _gather_kernel(n_id, memory.T, last_update)
    return (outt.T, last_out, jnp.array(0, dtype=jnp.int32))


# speedup vs baseline: 22.5029x; 1.0623x over previous
import jax
import jax.numpy as jnp
from jax import lax
from jax.experimental import pallas as pl
from jax.experimental.pallas import tpu as pltpu
from jax.experimental.pallas import tpu_sc as plsc

NUM_NODES = 1000000
MEMORY_DIM = 64
BATCH = 16384

_NC = 2
_NS = 16
_NW = _NC * _NS
_BPT = BATCH // _NW
_G = 32


_mesh = plsc.VectorSubcoreMesh(core_axis_name="c", subcore_axis_name="s")


@pl.kernel(
    out_type=(
        jax.ShapeDtypeStruct((MEMORY_DIM, BATCH), jnp.float32),
        jax.ShapeDtypeStruct((BATCH,), jnp.int32),
    ),
    mesh=_mesh,
    compiler_params=pltpu.CompilerParams(needs_layout_passes=False),
    scratch_types=[
        pltpu.VMEM((_BPT,), jnp.int32),
        pltpu.VMEM((MEMORY_DIM, _BPT), jnp.float32),
        pltpu.VMEM((MEMORY_DIM, 128), jnp.float32),
        pltpu.VMEM((MEMORY_DIM, 128), jnp.float32),
        pltpu.VMEM((MEMORY_DIM, 128), jnp.float32),
        pltpu.VMEM((MEMORY_DIM, 128), jnp.float32),
        pltpu.VMEM((MEMORY_DIM, 128), jnp.float32),
        pltpu.VMEM((MEMORY_DIM, 128), jnp.float32),
        pltpu.VMEM((MEMORY_DIM, 128), jnp.float32),
        pltpu.VMEM((MEMORY_DIM, 128), jnp.float32),
        pltpu.VMEM((_BPT,), jnp.int32),
        pltpu.SemaphoreType.DMA,
        pltpu.SemaphoreType.DMA,
        pltpu.SemaphoreType.DMA,
        pltpu.SemaphoreType.DMA,
        pltpu.SemaphoreType.DMA,
        pltpu.SemaphoreType.DMA,
        pltpu.SemaphoreType.DMA,
        pltpu.SemaphoreType.DMA,
        pltpu.SemaphoreType.DMA,
    ],
)
def _gather_kernel(n_id_hbm, memt_hbm, last_hbm, outt_hbm, last_out_hbm,
                   idx_v, stag_v, buf0_v, buf1_v, buf2_v, buf3_v,
                   buf4_v, buf5_v, buf6_v, buf7_v, last_v,
                   sem0, sem1, sem2, sem3, sem4, sem5, sem6, sem7, sem_last):
    wid = lax.axis_index("s") * _NC + lax.axis_index("c")
    base = wid * _BPT
    pltpu.sync_copy(n_id_hbm.at[pl.ds(base, _BPT)], idx_v)
    cp_last = pltpu.async_copy(last_hbm.at[idx_v], last_v, sem_last)

    bufs = (buf0_v, buf1_v, buf2_v, buf3_v, buf4_v, buf5_v, buf6_v, buf7_v)
    sems = (sem0, sem1, sem2, sem3, sem4, sem5, sem6, sem7)
    _D = len(bufs)
    lanes = lax.iota(jnp.int32, 16)
    jvecs = [lanes + 16 * k for k in range(MEMORY_DIM // 16)]

    def fetch(i, slot):
        blk = pl.multiple_of((i >> 7) << 7, 128)
        return pltpu.async_copy(
            memt_hbm.at[:, pl.ds(blk, 128)], bufs[slot], sems[slot])

    def serve(off, r, slot):
        offv = jnp.full((16,), off, dtype=jnp.int32)
        rv = jnp.full((16,), r, dtype=jnp.int32)
        for jv in jvecs:
            vals = plsc.load_gather(bufs[slot], [jv, offv])
            plsc.store_scatter(stag_v, [jv, rv], vals)

    def group(g):
        ivs = [plsc.load_gather(idx_v, [lanes + (g * _G + 16 * h)])
               for h in range(_G // 16)]
        offs = [iv & 127 for iv in ivs]
        copies = [None] * _D

        def idx_at(k):
            return ivs[k // 16][k % 16]

        def off_at(k):
            return offs[k // 16][k % 16]

        for k in range(_G):
            slot = k % _D
            if copies[slot] is not None:
                copies[slot].wait()
                serve(off_at(k - _D), g * _G + k - _D, slot)
            copies[slot] = fetch(idx_at(k), slot)
        for k in range(_G - _D, _G):
            slot = k % _D
            copies[slot].wait()
            serve(off_at(k), g * _G + k, slot)

    pl.loop(0, _BPT // _G)(group)

    cp_last.wait()
    pltpu.sync_copy(last_v, last_out_hbm.at[pl.ds(base, _BPT)])
    pltpu.sync_copy(stag_v, outt_hbm.at[:, pl.ds(base, _BPT)])


def kernel(n_id, memory, last_update):
    outt, last_out = _gather_kernel(n_id, memory.T, last_update)
    return (outt.T, last_out, jnp.array(0, dtype=jnp.int32))
